# baseline (speedup 1.0000x reference)
import functools

import jax
import jax.numpy as jnp
from jax import lax
from jax.experimental import pallas as pl
from jax.experimental.pallas import tpu as pltpu
from jax.experimental.pallas import tpu_sc as plsc

_B, _T, _EMB = 4096, 200, 64
_CHUNK = 128
_NBUF = 4
_LOOK = 2


def _sc_embed(x_flat, table, pos):
    info = plsc.get_sparse_core_info()
    nw = info.num_cores * info.num_subcores
    tok_per_w = (_B * _T) // nw
    n_chunks = tok_per_w // _CHUNK

    mesh = plsc.VectorSubcoreMesh(core_axis_name="c", subcore_axis_name="s")

    @functools.partial(
        pl.kernel,
        out_type=jax.ShapeDtypeStruct((_B * _T, _EMB), jnp.float32),
        mesh=mesh,
        scratch_types=[
            pltpu.VMEM((tok_per_w,), jnp.int32),
            pltpu.VMEM((2 * _T, _EMB), jnp.float32),
            pltpu.VMEM((_NBUF, _CHUNK, _EMB), jnp.float32),
        ]
        + [pltpu.SemaphoreType.DMA] * (2 * _NBUF),
        compiler_params=pltpu.CompilerParams(use_tc_tiling_on_sc=False),
    )
    def k(x_hbm, table_hbm, pos_hbm, out_hbm, idx_v, pos2_v, rows_v, *sems):
        sem_g = sems[:_NBUF]
        sem_o = sems[_NBUF:]
        wid = lax.axis_index("s") * info.num_cores + lax.axis_index("c")
        base = wid * tok_per_w

        pltpu.sync_copy(x_hbm.at[pl.ds(base, tok_per_w)], idx_v)
        pltpu.sync_copy(pos_hbm, pos2_v.at[pl.ds(0, _T)])
        pltpu.sync_copy(pos_hbm, pos2_v.at[pl.ds(_T, _T)])

        def gather_start(c, b):
            pltpu.async_copy(
                table_hbm.at[idx_v.at[pl.ds(c * _CHUNK, _CHUNK)]],
                rows_v.at[b],
                sem_g[b],
            )

        def gather_wait(c, b):
            pltpu.make_async_copy(
                table_hbm.at[idx_v.at[pl.ds(c * _CHUNK, _CHUNK)]],
                rows_v.at[b],
                sem_g[b],
            ).wait()

        def out_start(c, b):
            pltpu.async_copy(
                rows_v.at[b],
                out_hbm.at[pl.ds(base + c * _CHUNK, _CHUNK)],
                sem_o[b],
            )

        def out_wait(c, b):
            pltpu.make_async_copy(
                rows_v.at[b],
                out_hbm.at[pl.ds(base + c * _CHUNK, _CHUNK)],
                sem_o[b],
            ).wait()

        for c in range(_LOOK):
            gather_start(c, c % _NBUF)

        @pl.loop(0, n_chunks // _NBUF)
        def _grp(g):
            for b in range(_NBUF):
                c = g * _NBUF + b
                cc = c + _LOOK
                b2 = (b + _LOOK) % _NBUF

                @pl.when(cc < n_chunks)
                def _issue():
                    @pl.when(cc >= _NBUF)
                    def _drain():
                        out_wait(cc - _NBUF, b2)

                    gather_start(cc, b2)

                gather_wait(c, b)

                t0 = (c * _CHUNK) % _T
                rb = rows_v.at[b]

                @pl.loop(0, _CHUNK, unroll=2)
                def _add(i):
                    for s in range(_EMB // 16):
                        sl = pl.ds(s * 16, 16)
                        rb[i, sl] = rb[i, sl] + pos2_v[t0 + i, sl]

                out_start(c, b)

        for c in range(n_chunks - _NBUF, n_chunks):
            out_wait(c, c % _NBUF)

    return k(x_flat, table, pos)


def kernel(x, input_table, pos_table, positions):
    pos = jnp.take(pos_table, positions, axis=0)
    out = _sc_embed(x.reshape(-1).astype(jnp.int32), input_table, pos)
    return out.reshape(_B, _T, _EMB)
